# baseline (speedup 1.0000x reference)
import functools

import jax
import jax.numpy as jnp
from jax.experimental import pallas as pl

NQ = 8
NE = 1024
D = 256


def _rvq_kernel(x_ref, emb_ref, embt_ref, q_ref, codes_ref, loss_ref, *,
                blk, n_total):
    @pl.when(pl.program_id(0) == 0)
    def _():
        loss_ref[...] = jnp.zeros((1, 1), jnp.float32)

    r = x_ref[...]
    qsum = jnp.zeros_like(r)
    loss = jnp.float32(0.0)
    iota = jax.lax.broadcasted_iota(jnp.int32, (blk, NE), 1)
    idx_cols = []
    for i in range(NQ):
        emb_i = emb_ref[i]
        zn = jnp.sum(r * r, axis=1, keepdims=True)
        en = jnp.sum(emb_i * emb_i, axis=1, keepdims=True).reshape(1, NE)
        mm = jnp.dot(r.astype(jnp.bfloat16), embt_ref[i],
                     preferred_element_type=jnp.float32)
        dist = (zn + en) - 2.0 * mm
        mind = jnp.min(dist, axis=1, keepdims=True)
        idxc = jnp.min(jnp.where(dist == mind, iota, NE), axis=1,
                       keepdims=True)
        oh = (iota == idxc).astype(jnp.float32)
        q = jnp.dot(oh, emb_i, preferred_element_type=jnp.float32,
                    precision=jax.lax.Precision.HIGHEST)
        loss += 0.25 * (jnp.sum((q - r) ** 2) / jnp.float32(n_total))
        xq = r + (q - r)
        qsum = qsum + xq
        r = r - xq
        idx_cols.append(idxc)

    q_ref[...] = qsum
    codes_ref[...] = jnp.concatenate(idx_cols, axis=1)
    loss_ref[...] += jnp.full((1, 1), loss, jnp.float32)


def kernel(x, emb):
    B, T, Dd = x.shape
    n = B * T
    flat = x.reshape(n, Dd)
    embt = jnp.swapaxes(emb, 1, 2).astype(jnp.bfloat16)
    blk = 1152
    grid = (n // blk,)

    q, codes, loss = pl.pallas_call(
        functools.partial(_rvq_kernel, blk=blk, n_total=n * Dd),
        grid=grid,
        in_specs=[
            pl.BlockSpec((blk, Dd), lambda i: (i, 0)),
            pl.BlockSpec((NQ, NE, Dd), lambda i: (0, 0, 0)),
            pl.BlockSpec((NQ, Dd, NE), lambda i: (0, 0, 0)),
        ],
        out_specs=[
            pl.BlockSpec((blk, Dd), lambda i: (i, 0)),
            pl.BlockSpec((blk, NQ), lambda i: (i, 0)),
            pl.BlockSpec((1, 1), lambda i: (0, 0)),
        ],
        out_shape=[
            jax.ShapeDtypeStruct((n, Dd), jnp.float32),
            jax.ShapeDtypeStruct((n, NQ), jnp.int32),
            jax.ShapeDtypeStruct((1, 1), jnp.float32),
        ],
    )(flat, emb, embt)

    return q.reshape(B, T, Dd), loss[0, 0], codes.reshape(B, T, NQ)
